# baseline (speedup 1.0000x reference)
import functools

import jax
import jax.numpy as jnp
from jax.experimental import pallas as pl
from jax.experimental.pallas import tpu as pltpu

B = 4096
N = 2001
ROW_BLK = 512
PAIR_BLK = 512


def _row_reduce_kernel(out_ref, tgt_ref, pos_ref, neg_ref):
    x = out_ref[...]
    t = tgt_ref[...]
    pos_ref[...] = jnp.sum(jnp.where(t > 0, x, 0.0), axis=1, keepdims=True)
    neg_ref[...] = jnp.min(jnp.where(t == 0, x, jnp.inf), axis=1, keepdims=True)


def _pair_kernel(neg_ref, pos_ref, acc_ref):
    i = pl.program_id(0)
    diff = pos_ref[...] - neg_ref[...]
    s = jnp.sum(jnp.log(jax.nn.sigmoid(diff)))

    @pl.when(i == 0)
    def _():
        acc_ref[0, 0] = 0.0

    acc_ref[0, 0] += s


@jax.jit
def kernel(output, target):
    pos, neg = pl.pallas_call(
        _row_reduce_kernel,
        grid=(B // ROW_BLK,),
        in_specs=[
            pl.BlockSpec((ROW_BLK, N), lambda i: (i, 0)),
            pl.BlockSpec((ROW_BLK, N), lambda i: (i, 0)),
        ],
        out_specs=[
            pl.BlockSpec((ROW_BLK, 1), lambda i: (i, 0)),
            pl.BlockSpec((ROW_BLK, 1), lambda i: (i, 0)),
        ],
        out_shape=[
            jax.ShapeDtypeStruct((B, 1), jnp.float32),
            jax.ShapeDtypeStruct((B, 1), jnp.float32),
        ],
    )(output, target)

    pos_row = pos.reshape(1, B)
    total = pl.pallas_call(
        _pair_kernel,
        grid=(B // PAIR_BLK,),
        in_specs=[
            pl.BlockSpec((PAIR_BLK, 1), lambda i: (i, 0)),
            pl.BlockSpec((1, B), lambda i: (0, 0)),
        ],
        out_specs=pl.BlockSpec(memory_space=pltpu.SMEM),
        out_shape=jax.ShapeDtypeStruct((1, 1), jnp.float32),
    )(neg, pos_row)

    return -total[0, 0] / (B * B)


# speedup vs baseline: 1.0463x; 1.0463x over previous
import jax
import jax.numpy as jnp
from jax.experimental import pallas as pl
from jax.experimental.pallas import tpu as pltpu

B = 4096
N = 2001
ROW_BLK = 512
NBLK = B // ROW_BLK


def _fused_kernel(out_ref, tgt_ref, acc_ref, em_scr, ep_scr):
    k = pl.program_id(0)
    x = out_ref[...]
    t = tgt_ref[...]
    pos = jnp.sum(jnp.where(t > 0, x, 0.0), axis=1, keepdims=True)
    m = jnp.min(jnp.where(t == 0, x, jnp.inf), axis=1, keepdims=True)

    em = jnp.exp(m)
    ep_row = jnp.exp(-pos).reshape(1, ROW_BLK)

    em_scr[pl.ds(k * ROW_BLK, ROW_BLK), :] = em
    ep_scr[:, pl.ds(k * ROW_BLK, ROW_BLK)] = ep_row

    @pl.when(k == 0)
    def _():
        acc_ref[0, 0] = 0.0

    def body_a(j, s):
        ep_t = ep_scr[:, pl.ds(j * ROW_BLK, ROW_BLK)]
        return s + jnp.sum(jnp.log1p(em * ep_t))

    def body_b(i, s):
        em_t = em_scr[pl.ds(i * ROW_BLK, ROW_BLK), :]
        return s + jnp.sum(jnp.log1p(em_t * ep_row))

    s = jax.lax.fori_loop(0, k + 1, body_a, 0.0)
    s = jax.lax.fori_loop(0, k, body_b, s)
    acc_ref[0, 0] += s


@jax.jit
def kernel(output, target):
    total = pl.pallas_call(
        _fused_kernel,
        grid=(NBLK,),
        in_specs=[
            pl.BlockSpec((ROW_BLK, N), lambda i: (i, 0)),
            pl.BlockSpec((ROW_BLK, N), lambda i: (i, 0)),
        ],
        out_specs=pl.BlockSpec(memory_space=pltpu.SMEM),
        out_shape=jax.ShapeDtypeStruct((1, 1), jnp.float32),
        scratch_shapes=[
            pltpu.VMEM((B, 1), jnp.float32),
            pltpu.VMEM((1, B), jnp.float32),
        ],
    )(output, target)

    return total[0, 0] / (B * B)


# speedup vs baseline: 2.3784x; 2.2732x over previous
import jax
import jax.numpy as jnp
from jax.experimental import pallas as pl
from jax.experimental.pallas import tpu as pltpu

B = 4096
N = 2001
BLK = 512
NBLK = B // BLK


def _fused_kernel(out_ref, tgt_ref, acc_ref, em_scr, ep_scr):
    k = pl.program_id(0)
    x = out_ref[...]
    t = tgt_ref[...]
    pos = jnp.sum(jnp.where(t > 0, x, 0.0), axis=0, keepdims=True)
    m = jnp.min(jnp.where(t == 0, x, jnp.inf), axis=0, keepdims=True)

    ep_row = jnp.exp(-pos)
    em_row = jnp.exp(m)
    em_col = em_row.reshape(BLK, 1)

    ep_scr[:, pl.ds(k * BLK, BLK)] = ep_row
    em_scr[pl.ds(k * BLK, BLK), :] = em_col

    @pl.when(k == 0)
    def _():
        acc_ref[0, 0] = 0.0

    def body_a(j, s):
        ep_t = ep_scr[:, pl.ds(j * BLK, BLK)]
        return s + jnp.sum(jnp.log1p(em_col * ep_t))

    def body_b(i, s):
        em_t = em_scr[pl.ds(i * BLK, BLK), :]
        return s + jnp.sum(jnp.log1p(em_t * ep_row))

    s = jax.lax.fori_loop(0, k + 1, body_a, 0.0)
    s = jax.lax.fori_loop(0, k, body_b, s)
    acc_ref[0, 0] += s


@jax.jit
def kernel(output, target):
    total = pl.pallas_call(
        _fused_kernel,
        grid=(NBLK,),
        in_specs=[
            pl.BlockSpec((N, BLK), lambda i: (0, i)),
            pl.BlockSpec((N, BLK), lambda i: (0, i)),
        ],
        out_specs=pl.BlockSpec(memory_space=pltpu.SMEM),
        out_shape=jax.ShapeDtypeStruct((1, 1), jnp.float32),
        scratch_shapes=[
            pltpu.VMEM((B, 1), jnp.float32),
            pltpu.VMEM((1, B), jnp.float32),
        ],
    )(output.T, target.T)

    return total[0, 0] / (B * B)


# speedup vs baseline: 2.4554x; 1.0324x over previous
import jax
import jax.numpy as jnp
from jax.experimental import pallas as pl
from jax.experimental.pallas import tpu as pltpu

B = 4096
N = 2001
BLK = 1024
NBLK = B // BLK


def _fused_kernel(out_ref, tgt_ref, acc_ref, em_scr, ep_scr):
    k = pl.program_id(0)
    x = out_ref[...]
    t = tgt_ref[...]
    pos = jnp.sum(jnp.where(t > 0, x, 0.0), axis=0, keepdims=True)
    m = jnp.min(jnp.where(t == 0, x, jnp.inf), axis=0, keepdims=True)

    ep_row = jnp.exp(-pos)
    em_row = jnp.exp(m)
    em_col = em_row.reshape(BLK, 1)

    ep_scr[:, pl.ds(k * BLK, BLK)] = ep_row
    em_scr[pl.ds(k * BLK, BLK), :] = em_col

    @pl.when(k == 0)
    def _():
        acc_ref[0, 0] = 0.0

    def body_a(j, s):
        ep_t = ep_scr[:, pl.ds(j * BLK, BLK)]
        return s + jnp.sum(jnp.log1p(em_col * ep_t))

    def body_b(i, s):
        em_t = em_scr[pl.ds(i * BLK, BLK), :]
        return s + jnp.sum(jnp.log1p(em_t * ep_row))

    s = jax.lax.fori_loop(0, k + 1, body_a, 0.0)
    s = jax.lax.fori_loop(0, k, body_b, s)
    acc_ref[0, 0] += s


@jax.jit
def kernel(output, target):
    total = pl.pallas_call(
        _fused_kernel,
        grid=(NBLK,),
        in_specs=[
            pl.BlockSpec((N, BLK), lambda i: (0, i)),
            pl.BlockSpec((N, BLK), lambda i: (0, i)),
        ],
        out_specs=pl.BlockSpec(memory_space=pltpu.SMEM),
        out_shape=jax.ShapeDtypeStruct((1, 1), jnp.float32),
        scratch_shapes=[
            pltpu.VMEM((B, 1), jnp.float32),
            pltpu.VMEM((1, B), jnp.float32),
        ],
    )(output.T, target.T)

    return total[0, 0] / (B * B)


# speedup vs baseline: 2.9709x; 1.2099x over previous
import jax
import jax.numpy as jnp
from jax.experimental import pallas as pl
from jax.experimental.pallas import tpu as pltpu

B = 4096
N = 2001
BLK = 1024
NBLK = B // BLK


def _fused_kernel(out_ref, tgt_ref, acc_ref, em_scr, ep_scr):
    k = pl.program_id(0)
    x = out_ref[...]
    t = tgt_ref[...]
    pos = jnp.sum(x * t, axis=0, keepdims=True)
    m = jnp.min(jnp.where(t == 0, x, jnp.inf), axis=0, keepdims=True)

    ep_row = jnp.exp(-pos)
    em_row = jnp.exp(m)
    em_col = em_row.reshape(BLK, 1)

    ep_scr[:, pl.ds(k * BLK, BLK)] = ep_row
    em_scr[pl.ds(k * BLK, BLK), :] = em_col

    @pl.when(k == 0)
    def _():
        acc_ref[0, 0] = 0.0

    def body_a(j, s):
        ep_t = ep_scr[:, pl.ds(j * BLK, BLK)]
        return s + jnp.sum(jnp.log2(1.0 + em_col * ep_t))

    def body_b(i, s):
        em_t = em_scr[pl.ds(i * BLK, BLK), :]
        return s + jnp.sum(jnp.log2(1.0 + em_t * ep_row))

    s = jax.lax.fori_loop(0, k + 1, body_a, 0.0)
    s = jax.lax.fori_loop(0, k, body_b, s)
    acc_ref[0, 0] += s


@jax.jit
def kernel(output, target):
    total = pl.pallas_call(
        _fused_kernel,
        grid=(NBLK,),
        in_specs=[
            pl.BlockSpec((N, BLK), lambda i: (0, i)),
            pl.BlockSpec((N, BLK), lambda i: (0, i)),
        ],
        out_specs=pl.BlockSpec(memory_space=pltpu.SMEM),
        out_shape=jax.ShapeDtypeStruct((1, 1), jnp.float32),
        scratch_shapes=[
            pltpu.VMEM((B, 1), jnp.float32),
            pltpu.VMEM((1, B), jnp.float32),
        ],
    )(output.T, target.T)

    return total[0, 0] * (0.6931471805599453 / (B * B))
